# baseline (speedup 1.0000x reference)
import functools

import jax
import jax.numpy as jnp
from jax import lax
from jax.experimental import pallas as pl
from jax.experimental.pallas import tpu as pltpu

N = 10000
D_IN = 128
HID = 32
D_OUT = 16
NCHUNK = 4
CHUNK = N // NCHUNK

_DNT = (((0,), (1,)), ((), ()))


def _mlp(x_ref, w0_ref, b0_ref, w1_ref, b1_ref, w2_ref, b2_ref, o_hbm,
         ov, out_sems):
    for i in range(NCHUNK):
        xs = x_ref[pl.ds(i * CHUNK, CHUNK), :]
        h = jnp.dot(xs, w0_ref[...], preferred_element_type=jnp.float32)
        h = jnp.maximum(h + b0_ref[...], 0.0)
        h = jnp.dot(h, w1_ref[...], preferred_element_type=jnp.float32)
        h = jnp.maximum(h + b1_ref[...], 0.0)
        ot = lax.dot_general(w2_ref[...], h, _DNT,
                             preferred_element_type=jnp.float32)
        ov[i] = ot + b2_ref[...]
        pltpu.make_async_copy(ov.at[i], o_hbm.at[i], out_sems.at[i]).start()
    for i in range(NCHUNK):
        pltpu.make_async_copy(ov.at[i], o_hbm.at[i], out_sems.at[i]).wait()


@functools.partial(jax.jit, static_argnames=())
def kernel(x, weight, W0, b0, W1, b1, W2, b2, edge_index, batch):
    del weight, edge_index, batch
    b0r = b0.reshape(1, HID)
    b1r = b1.reshape(1, HID)
    b2c = b2.reshape(D_OUT, 1)
    vmem = pl.BlockSpec(memory_space=pltpu.MemorySpace.VMEM)
    packed = pl.pallas_call(
        _mlp,
        in_specs=[vmem] * 7,
        out_specs=pl.BlockSpec(memory_space=pl.ANY),
        out_shape=jax.ShapeDtypeStruct((NCHUNK, D_OUT, CHUNK), jnp.float32),
        scratch_shapes=[
            pltpu.VMEM((NCHUNK, D_OUT, CHUNK), jnp.float32),
            pltpu.SemaphoreType.DMA((NCHUNK,)),
        ],
    )(x, W0, b0r, W1, b1r, W2, b2c)
    return packed.transpose(0, 2, 1).reshape(N, D_OUT)


# speedup vs baseline: 1.1254x; 1.1254x over previous
import functools

import jax
import jax.numpy as jnp
from jax import lax
from jax.experimental import pallas as pl
from jax.experimental.pallas import tpu as pltpu

N = 10000
D_IN = 128
HID = 32
D_OUT = 16
NCHUNK = 8
CHUNK = 1280
LAST = N - (NCHUNK - 1) * CHUNK

_DNT = (((0,), (1,)), ((), ()))


def _mlp(x_ref, w0_ref, b0_ref, w1_ref, b1_ref, w2_ref, b2_ref, o_hbm,
         ov, out_sem):
    i = pl.program_id(0)
    h = jnp.dot(x_ref[...], w0_ref[...], preferred_element_type=jnp.float32)
    h = jnp.maximum(h + b0_ref[...], 0.0)
    h = jnp.dot(h, w1_ref[...], preferred_element_type=jnp.float32)
    h = jnp.maximum(h + b1_ref[...], 0.0)
    ot = lax.dot_general(w2_ref[...], h, _DNT,
                         preferred_element_type=jnp.float32)
    @pl.when(i < NCHUNK - 1)
    def _store():
        ov[:, pl.ds(i * CHUNK, CHUNK)] = ot + b2_ref[...]

    @pl.when(i == NCHUNK - 1)
    def _store_tail_and_flush():
        ov[:, pl.ds((NCHUNK - 1) * CHUNK, LAST)] = (
            ot[:, :LAST] + b2_ref[...]
        )
        pltpu.make_async_copy(ov, o_hbm, out_sem).start()
        pltpu.make_async_copy(ov, o_hbm, out_sem).wait()


@functools.partial(jax.jit, static_argnames=())
def kernel(x, weight, W0, b0, W1, b1, W2, b2, edge_index, batch):
    del weight, edge_index, batch
    b0r = b0.reshape(1, HID)
    b1r = b1.reshape(1, HID)
    b2c = b2.reshape(D_OUT, 1)
    full = lambda i: (0, 0)
    xh = pltpu.with_memory_space_constraint(x, pltpu.MemorySpace.HBM)
    packed = pl.pallas_call(
        _mlp,
        grid=(NCHUNK,),
        in_specs=[
            pl.BlockSpec((CHUNK, D_IN), lambda i: (i, 0)),
            pl.BlockSpec((D_IN, HID), full),
            pl.BlockSpec((1, HID), full),
            pl.BlockSpec((HID, HID), full),
            pl.BlockSpec((1, HID), full),
            pl.BlockSpec((HID, D_OUT), full),
            pl.BlockSpec((D_OUT, 1), full),
        ],
        out_specs=pl.BlockSpec(memory_space=pl.ANY),
        out_shape=jax.ShapeDtypeStruct((D_OUT, N), jnp.float32),
        scratch_shapes=[
            pltpu.VMEM((D_OUT, N), jnp.float32),
            pltpu.SemaphoreType.DMA,
        ],
        compiler_params=pltpu.CompilerParams(
            dimension_semantics=("arbitrary",),
        ),
    )(xh, W0, b0r, W1, b1r, W2, b2c)
    return packed.T


# speedup vs baseline: 1.3200x; 1.1729x over previous
import functools

import jax
import jax.numpy as jnp
from jax import lax
from jax.experimental import pallas as pl
from jax.experimental.pallas import tpu as pltpu

N = 10000
D_IN = 128
HID = 32
D_OUT = 16
NCHUNK = 8
NSEG = 4
CHUNK = 1280
LAST = N - (NCHUNK - 1) * CHUNK

_DNT = (((0,), (1,)), ((), ()))


def _mlp(x0_ref, x1_ref, x2_ref, x3_ref,
         w0_ref, b0_ref, w1_ref, b1_ref, w2_ref, b2_ref, o_hbm,
         ov, out_sem):
    t = pl.program_id(0)
    for s, x_ref in enumerate((x0_ref, x1_ref, x2_ref, x3_ref)):
        c = 2 * s + t
        h = jnp.dot(x_ref[...], w0_ref[...],
                    preferred_element_type=jnp.float32)
        h = jnp.maximum(h + b0_ref[...], 0.0)
        h = jnp.dot(h, w1_ref[...], preferred_element_type=jnp.float32)
        h = jnp.maximum(h + b1_ref[...], 0.0)
        ot = lax.dot_general(w2_ref[...], h, _DNT,
                             preferred_element_type=jnp.float32)
        ot = ot + b2_ref[...]
        if s < NSEG - 1:
            ov[:, pl.ds(c * CHUNK, CHUNK)] = ot
        else:
            @pl.when(t == 0)
            def _store_full():
                ov[:, pl.ds(c * CHUNK, CHUNK)] = ot

            @pl.when(t == 1)
            def _store_tail():
                ov[:, pl.ds((NCHUNK - 1) * CHUNK, LAST)] = ot[:, :LAST]

    @pl.when(t == 1)
    def _flush():
        pltpu.make_async_copy(ov, o_hbm, out_sem).start()
        pltpu.make_async_copy(ov, o_hbm, out_sem).wait()


@functools.partial(jax.jit, static_argnames=())
def kernel(x, weight, W0, b0, W1, b1, W2, b2, edge_index, batch):
    del weight, edge_index, batch
    b0r = b0.reshape(1, HID)
    b1r = b1.reshape(1, HID)
    b2c = b2.reshape(D_OUT, 1)
    full = lambda i: (0, 0)
    xh = pltpu.with_memory_space_constraint(x, pltpu.MemorySpace.HBM)
    packed = pl.pallas_call(
        _mlp,
        grid=(2,),
        in_specs=(
            [pl.BlockSpec((CHUNK, D_IN),
                          (lambda t, s=s: (2 * s + t, 0)))
             for s in range(NSEG)]
            + [
                pl.BlockSpec((D_IN, HID), full),
                pl.BlockSpec((1, HID), full),
                pl.BlockSpec((HID, HID), full),
                pl.BlockSpec((1, HID), full),
                pl.BlockSpec((HID, D_OUT), full),
                pl.BlockSpec((D_OUT, 1), full),
            ]
        ),
        out_specs=pl.BlockSpec(memory_space=pl.ANY),
        out_shape=jax.ShapeDtypeStruct((D_OUT, N), jnp.float32),
        scratch_shapes=[
            pltpu.VMEM((D_OUT, N), jnp.float32),
            pltpu.SemaphoreType.DMA,
        ],
        compiler_params=pltpu.CompilerParams(
            dimension_semantics=("arbitrary",),
        ),
    )(xh, xh, xh, xh, W0, b0r, W1, b1r, W2, b2c)
    return packed.T
